# baseline (speedup 1.0000x reference)
import functools

import jax
import jax.numpy as jnp
from jax import lax
from jax.experimental import pallas as pl
from jax.experimental.pallas import tpu as pltpu
from jax.experimental.pallas import tpu_sc as plsc

_B = 16384
_F = 8

_info = plsc.get_sparse_core_info()
_NC, _NS = _info.num_cores, _info.num_subcores
_NW = _NC * _NS
_BPW = _B // _NW
_CH = 128
_NCH = _BPW // _CH


def _sc_gather_body(user_table, track_table, users2d, tracks2d,
                    u_out, t_out, uidx, tidx, urows, trows, sem):
    wid = lax.axis_index("s") * _NC + lax.axis_index("c")
    base = wid * _BPW
    pltpu.sync_copy(users2d.at[wid], uidx)
    pltpu.sync_copy(tracks2d.at[wid], tidx)
    handles = []
    for j in range(_NCH):
        handles.append(pltpu.async_copy(
            user_table.at[uidx.at[j]], urows.at[pl.ds(j * _CH, _CH)], sem))
        handles.append(pltpu.async_copy(
            track_table.at[tidx.at[j]], trows.at[pl.ds(j * _CH, _CH)], sem))
    for h in handles:
        h.wait()
    pltpu.sync_copy(urows, u_out.at[pl.ds(base, _BPW)])
    pltpu.sync_copy(trows, t_out.at[pl.ds(base, _BPW)])


@functools.partial(
    pl.kernel,
    mesh=plsc.VectorSubcoreMesh(core_axis_name="c", subcore_axis_name="s"),
    out_type=[
        jax.ShapeDtypeStruct((_B, _F), jnp.float32),
        jax.ShapeDtypeStruct((_B, _F), jnp.float32),
    ],
    scratch_types=[
        pltpu.VMEM((_NCH, _CH), jnp.int32),
        pltpu.VMEM((_NCH, _CH), jnp.int32),
        pltpu.VMEM((_BPW, _F), jnp.float32),
        pltpu.VMEM((_BPW, _F), jnp.float32),
        pltpu.SemaphoreType.DMA,
    ],
    compiler_params=pltpu.CompilerParams(use_tc_tiling_on_sc=False),
)
def _sc_gather(user_table, track_table, users2d, tracks2d,
               u_out, t_out, uidx, tidx, urows, trows, sem):
    _sc_gather_body(user_table, track_table, users2d, tracks2d,
                    u_out, t_out, uidx, tidx, urows, trows, sem)


_BM = 2048


def _mlp_body(u_ref, t_ref, w1_ref, b1_ref, w2_ref, b2_ref, w3_ref, b3_ref,
              o_ref):
    h = jnp.dot(u_ref[...], w1_ref[0:_F, :], preferred_element_type=jnp.float32)
    h = h + jnp.dot(t_ref[...], w1_ref[_F:2 * _F, :],
                    preferred_element_type=jnp.float32)
    h = jnp.maximum(h + b1_ref[...], 0.0)
    h = jnp.maximum(jnp.dot(h, w2_ref[...], preferred_element_type=jnp.float32)
                    + b2_ref[...], 0.0)
    o = jnp.dot(h, w3_ref[...], preferred_element_type=jnp.float32) + b3_ref[...]
    o_ref[...] = 1.0 / (1.0 + jnp.exp(-o))


def kernel(users, tracks, user_table, track_table, W1, b1, W2, b2, W3, b3):
    users2d = users.reshape(_NW, _NCH, _CH)
    tracks2d = tracks.reshape(_NW, _NCH, _CH)
    u_emb, t_emb = _sc_gather(user_table, track_table, users2d, tracks2d)

    grid = (_B // _BM,)
    out = pl.pallas_call(
        _mlp_body,
        grid=grid,
        in_specs=[
            pl.BlockSpec((_BM, _F), lambda i: (i, 0)),
            pl.BlockSpec((_BM, _F), lambda i: (i, 0)),
            pl.BlockSpec((2 * _F, 64), lambda i: (0, 0)),
            pl.BlockSpec((1, 64), lambda i: (0, 0)),
            pl.BlockSpec((64, 32), lambda i: (0, 0)),
            pl.BlockSpec((1, 32), lambda i: (0, 0)),
            pl.BlockSpec((32, 1), lambda i: (0, 0)),
            pl.BlockSpec((1, 1), lambda i: (0, 0)),
        ],
        out_specs=pl.BlockSpec((_BM, 1), lambda i: (i, 0)),
        out_shape=jax.ShapeDtypeStruct((_B, 1), jnp.float32),
    )(u_emb, t_emb, W1, b1.reshape(1, 64), W2, b2.reshape(1, 32),
      W3, b3.reshape(1, 1))
    return out


# speedup vs baseline: 6.8969x; 6.8969x over previous
import functools

import jax
import jax.numpy as jnp
from jax import lax
from jax.experimental import pallas as pl
from jax.experimental.pallas import tpu as pltpu
from jax.experimental.pallas import tpu_sc as plsc

_B = 16384
_F = 8
_N = 1000000

_info = plsc.get_sparse_core_info()
_NC, _NS = _info.num_cores, _info.num_subcores
_NW = _NC * _NS
_BPW = _B // _NW
_G = 16
_NG = _BPW // _G
_SLAB = 128
_NCHUNK = _B // 128


def _sc_body(ut_ref, tt_ref, users_ref, tracks_ref, u_out, t_out,
             idx_v, slabs, outb, sems):
    wid = lax.axis_index("s") * _NC + lax.axis_index("c")
    base = wid * _BPW

    def fire(table, g, slot):
        iv = idx_v[pl.ds(g * _G, _G)]
        tv = lax.shift_left(lax.shift_right_logical(iv, 7), 7)
        for s in range(_G):
            col0 = pl.multiple_of(tv[s], 128)
            pltpu.async_copy(
                table.at[:, pl.ds(col0, _SLAB)], slabs.at[slot, s],
                sems.at[slot])

    def drain(table, slot):
        for s in range(_G):
            pltpu.make_async_copy(
                table.at[:, pl.ds(0, _SLAB)], slabs.at[slot, s],
                sems.at[slot]).wait()

    def extract(g, slot):
        lanes = lax.iota(jnp.int32, _G)
        cv = idx_v[pl.ds(g * _G, _G)] & (_SLAB - 1)
        blk = lax.shift_right_logical(g, 3)
        lane0 = lax.shift_left(g & 7, 4)
        for f in range(_F):
            acc = jnp.zeros((_G,), jnp.float32)
            for s in range(_G):
                v = slabs[slot, s, f, pl.ds(cv[s] - s, _G)]
                acc = jnp.where(lanes == s, v, acc)
            outb[blk, f, pl.ds(lane0, _G)] = acc

    def do_table(table, idx_hbm, out_hbm):
        pltpu.sync_copy(idx_hbm.at[pl.ds(base, _BPW)], idx_v)
        fire(table, 0, 0)

        def step(k, _):
            g0 = 2 * k
            fire(table, g0 + 1, 1)
            drain(table, 0)
            extract(g0, 0)

            @pl.when(g0 + 2 < _NG)
            def _():
                fire(table, g0 + 2, 0)

            drain(table, 1)
            extract(g0 + 1, 1)
            return ()

        lax.fori_loop(0, _NG // 2, step, (), unroll=False)
        for b in range(_BPW // 128):
            pltpu.sync_copy(outb.at[b], out_hbm.at[wid * (_BPW // 128) + b])

    do_table(ut_ref, users_ref, u_out)
    do_table(tt_ref, tracks_ref, t_out)


@functools.partial(
    pl.kernel,
    mesh=plsc.VectorSubcoreMesh(core_axis_name="c", subcore_axis_name="s"),
    out_type=[
        jax.ShapeDtypeStruct((_NCHUNK, _F, 128), jnp.float32),
        jax.ShapeDtypeStruct((_NCHUNK, _F, 128), jnp.float32),
    ],
    scratch_types=[
        pltpu.VMEM((_BPW,), jnp.int32),
        pltpu.VMEM((2, _G + 1, _F, _SLAB), jnp.float32),
        pltpu.VMEM((_BPW // 128, _F, 128), jnp.float32),
        pltpu.SemaphoreType.DMA((2,)),
    ],
)
def _sc_gather(ut_ref, tt_ref, users_ref, tracks_ref, u_out, t_out,
               idx_v, slabs, outb, sems):
    _sc_body(ut_ref, tt_ref, users_ref, tracks_ref, u_out, t_out,
             idx_v, slabs, outb, sems)


_CB = 16


def _mlp_body(u_ref, t_ref, w1_ref, b1_ref, w2_ref, b2_ref, w3_ref, b3_ref,
              o_ref):
    dn = (((0,), (0,)), ((), ()))
    w1u = w1_ref[0:_F, :]
    w1t = w1_ref[_F:2 * _F, :]
    for c in range(_CB):
        h = lax.dot_general(w1u, u_ref[c], dn,
                            preferred_element_type=jnp.float32)
        h = h + lax.dot_general(w1t, t_ref[c], dn,
                                preferred_element_type=jnp.float32)
        h = jnp.maximum(h + b1_ref[...], 0.0)
        h = lax.dot_general(w2_ref[...], h, dn,
                            preferred_element_type=jnp.float32)
        h = jnp.maximum(h + b2_ref[...], 0.0)
        o = lax.dot_general(w3_ref[...], h, dn,
                            preferred_element_type=jnp.float32) + b3_ref[...]
        o_ref[c] = 1.0 / (1.0 + jnp.exp(-o))


def kernel(users, tracks, user_table, track_table, W1, b1, W2, b2, W3, b3):
    u_emb, t_emb = _sc_gather(user_table.T, track_table.T, users, tracks)

    out3 = pl.pallas_call(
        _mlp_body,
        grid=(_NCHUNK // _CB,),
        in_specs=[
            pl.BlockSpec((_CB, _F, 128), lambda i: (i, 0, 0)),
            pl.BlockSpec((_CB, _F, 128), lambda i: (i, 0, 0)),
            pl.BlockSpec((2 * _F, 64), lambda i: (0, 0)),
            pl.BlockSpec((64, 1), lambda i: (0, 0)),
            pl.BlockSpec((64, 32), lambda i: (0, 0)),
            pl.BlockSpec((32, 1), lambda i: (0, 0)),
            pl.BlockSpec((32, 1), lambda i: (0, 0)),
            pl.BlockSpec((1, 1), lambda i: (0, 0)),
        ],
        out_specs=pl.BlockSpec((_CB, 1, 128), lambda i: (i, 0, 0)),
        out_shape=jax.ShapeDtypeStruct((_NCHUNK, 1, 128), jnp.float32),
    )(u_emb, t_emb, W1, b1.reshape(64, 1), W2, b2.reshape(32, 1),
      W3, b3.reshape(1, 1))
    return out3.reshape(_B, 1)


# speedup vs baseline: 8.9445x; 1.2969x over previous
import functools

import jax
import jax.numpy as jnp
from jax import lax
from jax.experimental import pallas as pl
from jax.experimental.pallas import tpu as pltpu
from jax.experimental.pallas import tpu_sc as plsc

_B = 16384
_F = 8
_N = 1000000

_info = plsc.get_sparse_core_info()
_NC, _NS = _info.num_cores, _info.num_subcores
_NW = _NC * _NS
_BPW = _B // _NW
_G = 16
_NG = _BPW // _G
_SLAB = 128
_NCHUNK = _B // 128


def _sc_body(ut_ref, tt_ref, users_ref, tracks_ref, u_out, t_out,
             idx_v, slabs, outb, sems):
    wid = lax.axis_index("s") * _NC + lax.axis_index("c")
    base = wid * _BPW

    def fire(table, g, slot):
        iv = idx_v[pl.ds(g * _G, _G)]
        tv = lax.shift_left(lax.shift_right_logical(iv, 7), 7)
        for s in range(_G):
            col0 = pl.multiple_of(tv[s], 128)
            pltpu.async_copy(
                table.at[:, pl.ds(col0, _SLAB)], slabs.at[slot, s],
                sems.at[slot])

    def drain(table, slot):
        for s in range(_G):
            pltpu.make_async_copy(
                table.at[:, pl.ds(0, _SLAB)], slabs.at[slot, s],
                sems.at[slot]).wait()

    def extract(g, slot):
        lanes = lax.iota(jnp.int32, _G)
        cv = idx_v[pl.ds(g * _G, _G)] & (_SLAB - 1)
        blk = lax.shift_right_logical(g, 3)
        lane0 = lax.shift_left(g & 7, 4)
        for f in range(_F):
            acc = jnp.zeros((_G,), jnp.float32)
            for s in range(_G):
                v = slabs[slot, s, f, pl.ds(cv[s] - s, _G)]
                acc = jnp.where(lanes == s, v, acc)
            outb[blk, f, pl.ds(lane0, _G)] = acc

    def do_table(table, idx_hbm, out_hbm):
        pltpu.sync_copy(idx_hbm.at[pl.ds(base, _BPW)], idx_v)
        fire(table, 0, 0)

        def step(k, _):
            g0 = 2 * k
            fire(table, g0 + 1, 1)
            drain(table, 0)
            extract(g0, 0)

            @pl.when(g0 + 2 < _NG)
            def _():
                fire(table, g0 + 2, 0)

            drain(table, 1)
            extract(g0 + 1, 1)
            return ()

        lax.fori_loop(0, _NG // 2, step, (), unroll=False)
        for b in range(_BPW // 128):
            pltpu.sync_copy(outb.at[b], out_hbm.at[wid * (_BPW // 128) + b])

    do_table(ut_ref, users_ref, u_out)
    do_table(tt_ref, tracks_ref, t_out)


@functools.partial(
    pl.kernel,
    mesh=plsc.VectorSubcoreMesh(core_axis_name="c", subcore_axis_name="s"),
    out_type=[
        jax.ShapeDtypeStruct((_NCHUNK, _F, 128), jnp.float32),
        jax.ShapeDtypeStruct((_NCHUNK, _F, 128), jnp.float32),
    ],
    scratch_types=[
        pltpu.VMEM((_BPW,), jnp.int32),
        pltpu.VMEM((2, _G + 1, _F, _SLAB), jnp.float32),
        pltpu.VMEM((_BPW // 128, _F, 128), jnp.float32),
        pltpu.SemaphoreType.DMA((2,)),
    ],
)
def _sc_gather(ut_ref, tt_ref, users_ref, tracks_ref, u_out, t_out,
               idx_v, slabs, outb, sems):
    _sc_body(ut_ref, tt_ref, users_ref, tracks_ref, u_out, t_out,
             idx_v, slabs, outb, sems)


_CB = 16


def _mlp_body(u_ref, t_ref, w1_ref, b1_ref, w2_ref, b2_ref, w3_ref, b3_ref,
              o_ref):
    dn = (((0,), (0,)), ((), ()))
    h = lax.dot_general(w1_ref[0:_F, :], u_ref[...], dn,
                        preferred_element_type=jnp.float32)
    h = h + lax.dot_general(w1_ref[_F:2 * _F, :], t_ref[...], dn,
                            preferred_element_type=jnp.float32)
    h = jnp.maximum(h + b1_ref[...], 0.0)
    h = lax.dot_general(w2_ref[...], h, dn, preferred_element_type=jnp.float32)
    h = jnp.maximum(h + b2_ref[...], 0.0)
    o = lax.dot_general(w3_ref[...], h, dn,
                        preferred_element_type=jnp.float32) + b3_ref[...]
    o_ref[...] = 1.0 / (1.0 + jnp.exp(-o))


def kernel(users, tracks, user_table, track_table, W1, b1, W2, b2, W3, b3):
    u_emb, t_emb = _sc_gather(user_table.T, track_table.T, users, tracks)
    u2 = jnp.transpose(u_emb, (1, 0, 2)).reshape(_F, _B)
    t2 = jnp.transpose(t_emb, (1, 0, 2)).reshape(_F, _B)

    _BN = 2048
    out2 = pl.pallas_call(
        _mlp_body,
        grid=(_B // _BN,),
        in_specs=[
            pl.BlockSpec((_F, _BN), lambda i: (0, i)),
            pl.BlockSpec((_F, _BN), lambda i: (0, i)),
            pl.BlockSpec((2 * _F, 64), lambda i: (0, 0)),
            pl.BlockSpec((64, 1), lambda i: (0, 0)),
            pl.BlockSpec((64, 32), lambda i: (0, 0)),
            pl.BlockSpec((32, 1), lambda i: (0, 0)),
            pl.BlockSpec((32, 1), lambda i: (0, 0)),
            pl.BlockSpec((1, 1), lambda i: (0, 0)),
        ],
        out_specs=pl.BlockSpec((1, _BN), lambda i: (0, i)),
        out_shape=jax.ShapeDtypeStruct((1, _B), jnp.float32),
    )(u2, t2, W1, b1.reshape(64, 1), W2, b2.reshape(32, 1),
      W3, b3.reshape(1, 1))
    return out2.reshape(_B, 1)


# speedup vs baseline: 9.8466x; 1.1009x over previous
import functools

import jax
import jax.numpy as jnp
from jax import lax
from jax.experimental import pallas as pl
from jax.experimental.pallas import tpu as pltpu
from jax.experimental.pallas import tpu_sc as plsc

_B = 16384
_F = 8
_N = 1000000

_info = plsc.get_sparse_core_info()
_NC, _NS = _info.num_cores, _info.num_subcores
_NW = _NC * _NS
_BPW = _B // _NW
_G = 16
_NG = _BPW // _G
_SLAB = 128
_NBUF = 4
_NCHUNK = _B // 128


def _sc_body(ut_ref, tt_ref, users_ref, tracks_ref, u_out, t_out,
             idx_v, slabs, outb, sems):
    wid = lax.axis_index("s") * _NC + lax.axis_index("c")
    base = wid * _BPW

    def fire(table, g, slot):
        iv = idx_v[pl.ds(g * _G, _G)]
        tv = lax.shift_left(lax.shift_right_logical(iv, 7), 7)
        for s in range(_G):
            col0 = pl.multiple_of(tv[s], 128)
            pltpu.async_copy(
                table.at[:, pl.ds(col0, _SLAB)], slabs.at[slot, s],
                sems.at[slot])

    def drain(table, slot):
        for s in range(_G):
            pltpu.make_async_copy(
                table.at[:, pl.ds(0, _SLAB)], slabs.at[slot, s],
                sems.at[slot]).wait()

    def extract(g, slot):
        lanes = lax.iota(jnp.int32, _G)
        cv = idx_v[pl.ds(g * _G, _G)] & (_SLAB - 1)
        blk = lax.shift_right_logical(g, 3)
        lane0 = lax.shift_left(g & 7, 4)
        zero = jnp.zeros((_G,), jnp.float32)
        for f in range(_F):
            parts = [
                jnp.where(lanes == s,
                          slabs[slot, s, f, pl.ds(cv[s] - s, _G)], zero)
                for s in range(_G)
            ]
            while len(parts) > 1:
                parts = [a + b for a, b in zip(parts[::2], parts[1::2])]
            outb[blk, f, pl.ds(lane0, _G)] = parts[0]

    def do_table(table, idx_hbm, out_hbm):
        pltpu.sync_copy(idx_hbm.at[pl.ds(base, _BPW)], idx_v)
        for j in range(_NBUF):
            fire(table, j, j)

        def step(k, _):
            for j in range(_NBUF):
                g = _NBUF * k + j
                drain(table, j)
                extract(g, j)

                @pl.when(g + _NBUF < _NG)
                def _():
                    fire(table, g + _NBUF, j)
            return ()

        lax.fori_loop(0, _NG // _NBUF, step, (), unroll=False)
        for b in range(_BPW // 128):
            pltpu.sync_copy(outb.at[b], out_hbm.at[wid * (_BPW // 128) + b])

    do_table(ut_ref, users_ref, u_out)
    do_table(tt_ref, tracks_ref, t_out)


@functools.partial(
    pl.kernel,
    mesh=plsc.VectorSubcoreMesh(core_axis_name="c", subcore_axis_name="s"),
    out_type=[
        jax.ShapeDtypeStruct((_NCHUNK, _F, 128), jnp.float32),
        jax.ShapeDtypeStruct((_NCHUNK, _F, 128), jnp.float32),
    ],
    scratch_types=[
        pltpu.VMEM((_BPW,), jnp.int32),
        pltpu.VMEM((_NBUF, _G + 1, _F, _SLAB), jnp.float32),
        pltpu.VMEM((_BPW // 128, _F, 128), jnp.float32),
        pltpu.SemaphoreType.DMA((_NBUF,)),
    ],
)
def _sc_gather(ut_ref, tt_ref, users_ref, tracks_ref, u_out, t_out,
               idx_v, slabs, outb, sems):
    _sc_body(ut_ref, tt_ref, users_ref, tracks_ref, u_out, t_out,
             idx_v, slabs, outb, sems)


_CB = 16


def _mlp_body(u_ref, t_ref, w1_ref, b1_ref, w2_ref, b2_ref, w3_ref, b3_ref,
              o_ref):
    dn = (((0,), (0,)), ((), ()))
    h = lax.dot_general(w1_ref[0:_F, :], u_ref[...], dn,
                        preferred_element_type=jnp.float32)
    h = h + lax.dot_general(w1_ref[_F:2 * _F, :], t_ref[...], dn,
                            preferred_element_type=jnp.float32)
    h = jnp.maximum(h + b1_ref[...], 0.0)
    h = lax.dot_general(w2_ref[...], h, dn, preferred_element_type=jnp.float32)
    h = jnp.maximum(h + b2_ref[...], 0.0)
    o = lax.dot_general(w3_ref[...], h, dn,
                        preferred_element_type=jnp.float32) + b3_ref[...]
    o_ref[...] = 1.0 / (1.0 + jnp.exp(-o))


def kernel(users, tracks, user_table, track_table, W1, b1, W2, b2, W3, b3):
    u_emb, t_emb = _sc_gather(user_table.T, track_table.T, users, tracks)
    u2 = jnp.transpose(u_emb, (1, 0, 2)).reshape(_F, _B)
    t2 = jnp.transpose(t_emb, (1, 0, 2)).reshape(_F, _B)

    _BN = 2048
    out2 = pl.pallas_call(
        _mlp_body,
        grid=(_B // _BN,),
        in_specs=[
            pl.BlockSpec((_F, _BN), lambda i: (0, i)),
            pl.BlockSpec((_F, _BN), lambda i: (0, i)),
            pl.BlockSpec((2 * _F, 64), lambda i: (0, 0)),
            pl.BlockSpec((64, 1), lambda i: (0, 0)),
            pl.BlockSpec((64, 32), lambda i: (0, 0)),
            pl.BlockSpec((32, 1), lambda i: (0, 0)),
            pl.BlockSpec((32, 1), lambda i: (0, 0)),
            pl.BlockSpec((1, 1), lambda i: (0, 0)),
        ],
        out_specs=pl.BlockSpec((1, _BN), lambda i: (0, i)),
        out_shape=jax.ShapeDtypeStruct((1, _B), jnp.float32),
    )(u2, t2, W1, b1.reshape(64, 1), W2, b2.reshape(32, 1),
      W3, b3.reshape(1, 1))
    return out2.reshape(_B, 1)
